# Initial kernel scaffold; baseline (speedup 1.0000x reference)
import jax
import jax.numpy as jnp
from jax.experimental import pallas as pl


def kernel(positions, pe):
    raise NotImplementedError("write your pallas kernel here")


# speedup vs baseline: 1.9867x; 1.9867x over previous
import functools

import jax
import jax.numpy as jnp
from jax import lax
from jax.experimental import pallas as pl
from jax.experimental.pallas import tpu as pltpu, tpu_sc as plsc

D_MODEL = 1024
TOTAL = 4 * 8192

_info = plsc.get_sparse_core_info()
NUM_WORKERS = _info.num_cores * _info.num_subcores
B_PER_W = TOTAL // NUM_WORKERS
CHUNK = 32
N_CHUNKS = B_PER_W // CHUNK


def _gather_kernel(pe_hbm, idx_hbm, out_hbm, idx_v, rows_v, gsem):
    wid = lax.axis_index("s") * _info.num_cores + lax.axis_index("c")
    base = pl.multiple_of(wid * B_PER_W, B_PER_W)
    pltpu.sync_copy(idx_hbm.at[pl.ds(base, B_PER_W)], idx_v)

    def body(c, _):
        off = pl.multiple_of(c * CHUNK, CHUNK)
        idx_slice = idx_v.at[pl.ds(off, CHUNK)]
        pltpu.async_copy(pe_hbm.at[idx_slice], rows_v, gsem).wait()
        pltpu.sync_copy(rows_v, out_hbm.at[pl.ds(base + off, CHUNK)])
        return 0

    lax.fori_loop(0, N_CHUNKS, body, 0)


@jax.jit
def _gather(pe, idx_flat):
    mesh = plsc.VectorSubcoreMesh(core_axis_name="c", subcore_axis_name="s")
    run = functools.partial(
        pl.kernel,
        mesh=mesh,
        out_type=jax.ShapeDtypeStruct((TOTAL, D_MODEL), jnp.float32),
        scratch_types=[
            pltpu.VMEM((B_PER_W,), jnp.int32),
            pltpu.VMEM((CHUNK, D_MODEL), jnp.float32),
            pltpu.SemaphoreType.DMA,
        ],
    )(_gather_kernel)
    return run(pe, idx_flat)


def kernel(positions, pe):
    idx_flat = positions.reshape(-1)
    out = _gather(pe, idx_flat)
    return out.reshape(positions.shape + (D_MODEL,))


# speedup vs baseline: 2.2574x; 1.1363x over previous
import functools

import jax
import jax.numpy as jnp
from jax import lax
from jax.experimental import pallas as pl
from jax.experimental.pallas import tpu as pltpu, tpu_sc as plsc

D_MODEL = 1024
TOTAL = 4 * 8192

_info = plsc.get_sparse_core_info()
NUM_WORKERS = _info.num_cores * _info.num_subcores
B_PER_W = TOTAL // NUM_WORKERS
CHUNK = 32
N_CHUNKS = B_PER_W // CHUNK
N_STEPS = N_CHUNKS // 2


def _gather_kernel(pe_hbm, idx_hbm, out_hbm, idx_v, rows0, rows1,
                   gsem0, gsem1, ssem0, ssem1):
    wid = lax.axis_index("s") * _info.num_cores + lax.axis_index("c")
    base = pl.multiple_of(wid * B_PER_W, B_PER_W)
    pltpu.sync_copy(idx_hbm.at[pl.ds(base, B_PER_W)], idx_v)

    bufs = (rows0, rows1)
    gsems = (gsem0, gsem1)
    ssems = (ssem0, ssem1)

    def gstart(c, b):
        off = pl.multiple_of(c * CHUNK, CHUNK)
        pltpu.make_async_copy(
            pe_hbm.at[idx_v.at[pl.ds(off, CHUNK)]], bufs[b], gsems[b]
        ).start()

    def gwait(b):
        pltpu.make_async_copy(
            pe_hbm.at[idx_v.at[pl.ds(0, CHUNK)]], bufs[b], gsems[b]
        ).wait()

    def sstart(c, b):
        off = pl.multiple_of(c * CHUNK, CHUNK)
        pltpu.make_async_copy(
            bufs[b], out_hbm.at[pl.ds(base + off, CHUNK)], ssems[b]
        ).start()

    def swait(b):
        pltpu.make_async_copy(
            bufs[b], out_hbm.at[pl.ds(base, CHUNK)], ssems[b]
        ).wait()

    gstart(0, 0)
    gstart(1, 1)

    def body(s, _):
        c0 = s * 2
        for b in range(2):
            gwait(b)
            sstart(c0 + b, b)

        @pl.when(s + 1 < N_STEPS)
        def _():
            for b in range(2):
                swait(b)
                gstart(c0 + 2 + b, b)

        return 0

    lax.fori_loop(0, N_STEPS, body, 0)
    swait(0)
    swait(1)


@jax.jit
def _gather(pe, idx_flat):
    mesh = plsc.VectorSubcoreMesh(core_axis_name="c", subcore_axis_name="s")
    run = functools.partial(
        pl.kernel,
        mesh=mesh,
        out_type=jax.ShapeDtypeStruct((TOTAL, D_MODEL), jnp.float32),
        scratch_types=[
            pltpu.VMEM((B_PER_W,), jnp.int32),
            pltpu.VMEM((CHUNK, D_MODEL), jnp.float32),
            pltpu.VMEM((CHUNK, D_MODEL), jnp.float32),
            pltpu.SemaphoreType.DMA,
            pltpu.SemaphoreType.DMA,
            pltpu.SemaphoreType.DMA,
            pltpu.SemaphoreType.DMA,
        ],
    )(_gather_kernel)
    return run(pe, idx_flat)


def kernel(positions, pe):
    idx_flat = positions.reshape(-1)
    out = _gather(pe, idx_flat)
    return out.reshape(positions.shape + (D_MODEL,))


# speedup vs baseline: 2.3354x; 1.0345x over previous
import functools

import jax
import jax.numpy as jnp
from jax import lax
from jax.experimental import pallas as pl
from jax.experimental.pallas import tpu as pltpu, tpu_sc as plsc

D_MODEL = 1024
TOTAL = 4 * 8192

_info = plsc.get_sparse_core_info()
NUM_WORKERS = _info.num_cores * _info.num_subcores
B_PER_W = TOTAL // NUM_WORKERS
CHUNK = 16
NBUF = 4
N_CHUNKS = B_PER_W // CHUNK
N_STEPS = N_CHUNKS // NBUF


def _gather_kernel(pe_hbm, idx_hbm, out_hbm, idx_v, rows0, rows1, rows2, rows3,
                   gsem0, gsem1, gsem2, gsem3, ssem0, ssem1, ssem2, ssem3):
    wid = lax.axis_index("s") * _info.num_cores + lax.axis_index("c")
    base = pl.multiple_of(wid * B_PER_W, B_PER_W)
    pltpu.sync_copy(idx_hbm.at[pl.ds(base, B_PER_W)], idx_v)

    bufs = (rows0, rows1, rows2, rows3)
    gsems = (gsem0, gsem1, gsem2, gsem3)
    ssems = (ssem0, ssem1, ssem2, ssem3)

    def gstart(c, b):
        off = pl.multiple_of(c * CHUNK, CHUNK)
        pltpu.make_async_copy(
            pe_hbm.at[idx_v.at[pl.ds(off, CHUNK)]], bufs[b], gsems[b]
        ).start()

    def gwait(b):
        pltpu.make_async_copy(
            pe_hbm.at[idx_v.at[pl.ds(0, CHUNK)]], bufs[b], gsems[b]
        ).wait()

    def sstart(c, b):
        off = pl.multiple_of(c * CHUNK, CHUNK)
        pltpu.make_async_copy(
            bufs[b], out_hbm.at[pl.ds(base + off, CHUNK)], ssems[b]
        ).start()

    def swait(b):
        pltpu.make_async_copy(
            bufs[b], out_hbm.at[pl.ds(base, CHUNK)], ssems[b]
        ).wait()

    for b in range(NBUF):
        gstart(b, b)

    def body(s, _):
        c0 = s * NBUF
        for b in range(NBUF):
            gwait(b)
            sstart(c0 + b, b)

        @pl.when(s + 1 < N_STEPS)
        def _():
            for b in range(NBUF):
                swait(b)
                gstart(c0 + NBUF + b, b)

        return 0

    lax.fori_loop(0, N_STEPS, body, 0)
    for b in range(NBUF):
        swait(b)


@jax.jit
def _gather(pe, idx_flat):
    mesh = plsc.VectorSubcoreMesh(core_axis_name="c", subcore_axis_name="s")
    run = functools.partial(
        pl.kernel,
        mesh=mesh,
        out_type=jax.ShapeDtypeStruct((TOTAL, D_MODEL), jnp.float32),
        scratch_types=(
            [pltpu.VMEM((B_PER_W,), jnp.int32)]
            + [pltpu.VMEM((CHUNK, D_MODEL), jnp.float32)] * NBUF
            + [pltpu.SemaphoreType.DMA] * (2 * NBUF)
        ),
    )(_gather_kernel)
    return run(pe, idx_flat)


def kernel(positions, pe):
    idx_flat = positions.reshape(-1)
    out = _gather(pe, idx_flat)
    return out.reshape(positions.shape + (D_MODEL,))


# speedup vs baseline: 2.3814x; 1.0197x over previous
import functools

import jax
import jax.numpy as jnp
from jax import lax
from jax.experimental import pallas as pl
from jax.experimental.pallas import tpu as pltpu, tpu_sc as plsc

D_MODEL = 1024
TOTAL = 4 * 8192

_info = plsc.get_sparse_core_info()
NUM_WORKERS = _info.num_cores * _info.num_subcores
B_PER_W = TOTAL // NUM_WORKERS
CHUNK = 16
NBUF = 4
N_CHUNKS = B_PER_W // CHUNK
N_STEPS = N_CHUNKS // NBUF


def _gather_kernel(pe_hbm, idx_hbm, out_hbm, idx_v, rows0, rows1, rows2, rows3,
                   gsem0, gsem1, gsem2, gsem3, ssem0, ssem1, ssem2, ssem3):
    wid = lax.axis_index("s") * _info.num_cores + lax.axis_index("c")
    base = pl.multiple_of(wid * B_PER_W, B_PER_W)
    pltpu.sync_copy(idx_hbm.at[pl.ds(base, B_PER_W)], idx_v)

    bufs = (rows0, rows1, rows2, rows3)
    gsems = (gsem0, gsem1, gsem2, gsem3)
    ssems = (ssem0, ssem1, ssem2, ssem3)

    def gstart(c, b):
        off = pl.multiple_of(c * CHUNK, CHUNK)
        pltpu.make_async_copy(
            pe_hbm.at[idx_v.at[pl.ds(off, CHUNK)]], bufs[b], gsems[b]
        ).start()

    def gwait(b):
        pltpu.make_async_copy(
            pe_hbm.at[idx_v.at[pl.ds(0, CHUNK)]], bufs[b], gsems[b]
        ).wait()

    def sstart(c, b):
        off = pl.multiple_of(c * CHUNK, CHUNK)
        pltpu.make_async_copy(
            bufs[b], out_hbm.at[pl.ds(base + off, CHUNK)], ssems[b]
        ).start()

    def swait(b):
        pltpu.make_async_copy(
            bufs[b], out_hbm.at[pl.ds(base, CHUNK)], ssems[b]
        ).wait()

    for b in range(NBUF - 1):
        gstart(b, b)

    for b in range(NBUF):
        gwait(b)
        sstart(b, b)
        pb = (b - 1) % NBUF
        if b > 0:
            swait(pb)
        gstart(b + NBUF - 1, pb)

    def body(s, _):
        for b in range(NBUF):
            c = s * NBUF + b
            gwait(b)
            sstart(c, b)
            pb = (b - 1) % NBUF

            @pl.when(c + NBUF - 1 < N_CHUNKS)
            def _():
                swait(pb)
                gstart(c + NBUF - 1, pb)

        return 0

    lax.fori_loop(1, N_STEPS, body, 0)
    for b in range(NBUF):
        swait(b)


@jax.jit
def _gather(pe, idx_flat):
    mesh = plsc.VectorSubcoreMesh(core_axis_name="c", subcore_axis_name="s")
    run = functools.partial(
        pl.kernel,
        mesh=mesh,
        out_type=jax.ShapeDtypeStruct((TOTAL, D_MODEL), jnp.float32),
        scratch_types=(
            [pltpu.VMEM((B_PER_W,), jnp.int32)]
            + [pltpu.VMEM((CHUNK, D_MODEL), jnp.float32)] * NBUF
            + [pltpu.SemaphoreType.DMA] * (2 * NBUF)
        ),
    )(_gather_kernel)
    return run(pe, idx_flat)


def kernel(positions, pe):
    idx_flat = positions.reshape(-1)
    out = _gather(pe, idx_flat)
    return out.reshape(positions.shape + (D_MODEL,))
